# Initial kernel scaffold; baseline (speedup 1.0000x reference)
import jax
import jax.numpy as jnp
from jax.experimental import pallas as pl


def kernel(features, neighborhood_source, neighborhood_target, W, b):
    raise NotImplementedError("write your pallas kernel here")


# speedup vs baseline: 21.6562x; 21.6562x over previous
import functools
import jax
import jax.numpy as jnp
from jax import lax
from jax.experimental import pallas as pl
from jax.experimental.pallas import tpu as pltpu
from jax.experimental.pallas import tpu_sc as plsc

NEG_SLOPE = 0.3
NC, NS = 2, 16
NW = NC * NS
CHUNK = 128


def _matmul_body(x_ref, w_ref, a_ref, b_ref, *, c_out):
    y = jnp.dot(x_ref[...], w_ref[...], preferred_element_type=jnp.float32)
    a_ref[...] = y[:, :c_out]
    b_ref[...] = y[:, c_out:]


def _finalize_body(a_ref, bias_ref, s_ref, c_ref, o_ref):
    s = s_ref[0] + s_ref[1]
    cnt = c_ref[0, :, 0:1] + c_ref[1, :, 0:1]
    pre = jnp.where(cnt > 0.0, a_ref[...] + bias_ref[...], 0.0)
    pre = pre + s / jnp.maximum(cnt, 1.0)
    o_ref[...] = jnp.where(pre >= 0.0, pre, NEG_SLOPE * pre)


def _edge_body(b_hbm, src_hbm, dst_hbm, ones_hbm, z32_hbm, z8_hbm,
               s_out, c_out,
               b_sp, s_sp, c_sp,
               src_v, dst_v, rows_v, ones_v, zb32_v, zb8_v, sem,
               *, n_pad, n_chunks, c_out_dim):
    rows_per_tile = n_pad // NS
    c = lax.axis_index("c")
    s = lax.axis_index("s")
    wid = c * NS + s
    base = s * rows_per_tile
    sl = pl.ds(base, rows_per_tile)

    pltpu.sync_copy(z32_hbm, zb32_v)
    pltpu.sync_copy(z8_hbm, zb8_v)
    pltpu.sync_copy(zb32_v, s_sp.at[sl])
    pltpu.sync_copy(zb8_v, c_sp.at[sl])
    pltpu.sync_copy(b_hbm.at[sl], b_sp.at[sl])

    pltpu.sync_copy(src_hbm.at[wid], src_v)
    pltpu.sync_copy(dst_hbm.at[wid], dst_v)
    pltpu.sync_copy(ones_hbm, ones_v)
    plsc.subcore_barrier()

    def step(j, carry):
        pltpu.async_copy(b_sp.at[src_v.at[j]], rows_v, sem).wait()
        pltpu.sync_copy(rows_v, s_sp.at[dst_v.at[j]], add=True)
        pltpu.sync_copy(ones_v, c_sp.at[dst_v.at[j]], add=True)
        return carry

    lax.fori_loop(0, n_chunks, step, 0)
    plsc.subcore_barrier()

    pltpu.sync_copy(s_sp.at[sl], s_out.at[c, sl])
    pltpu.sync_copy(c_sp.at[sl], c_out.at[c, sl])


def kernel(features, neighborhood_source, neighborhood_target, W, b):
    n, c_in = features.shape
    e = neighborhood_source.shape[0]
    c_out = W.shape[0]

    n_pad = ((n + NS * 8 - 1) // (NS * 8)) * (NS * 8)
    n_pad = max(n_pad, ((n + 255) // 256) * 256)
    rows_per_tile = n_pad // NS
    epw = -(-e // NW)
    n_chunks = -(-epw // CHUNK)
    e_pad = NW * n_chunks * CHUNK

    w_loc, w_nb = W[:, :c_in], W[:, c_in:]
    w_cat = jnp.concatenate([w_loc - w_nb, w_nb], axis=0).T
    f_pad = jnp.pad(features, ((0, n_pad - n), (0, 0)))

    pad_cnt = e_pad - e
    pad_iota = jnp.arange(pad_cnt, dtype=jnp.int32)
    src_pad = jnp.concatenate([neighborhood_source,
                               pad_iota % jnp.int32(n)]).reshape(NW, n_chunks, CHUNK)
    dst_pad = jnp.concatenate([neighborhood_target,
                               jnp.int32(n) + pad_iota % jnp.int32(n_pad - n)]
                              ).reshape(NW, n_chunks, CHUNK)

    ones_rows = jnp.ones((CHUNK, 8), jnp.float32)
    z32 = jnp.zeros((rows_per_tile, c_out), jnp.float32)
    z8 = jnp.zeros((rows_per_tile, 8), jnp.float32)

    a_nodes, b_nodes = pl.pallas_call(
        functools.partial(_matmul_body, c_out=c_out),
        out_shape=(jax.ShapeDtypeStruct((n_pad, c_out), jnp.float32),
                   jax.ShapeDtypeStruct((n_pad, c_out), jnp.float32)),
    )(f_pad, w_cat)

    mesh = plsc.VectorSubcoreMesh(core_axis_name="c", subcore_axis_name="s",
                                  num_cores=NC, num_subcores=NS)
    edge_kernel = pl.kernel(
        functools.partial(_edge_body, n_pad=n_pad, n_chunks=n_chunks,
                          c_out_dim=c_out),
        out_type=(jax.ShapeDtypeStruct((NC, n_pad, c_out), jnp.float32),
                  jax.ShapeDtypeStruct((NC, n_pad, 8), jnp.float32)),
        mesh=mesh,
        scratch_types=[
            pltpu.VMEM_SHARED((n_pad, c_out), jnp.float32),
            pltpu.VMEM_SHARED((n_pad, c_out), jnp.float32),
            pltpu.VMEM_SHARED((n_pad, 8), jnp.float32),
            pltpu.VMEM((n_chunks, CHUNK), jnp.int32),
            pltpu.VMEM((n_chunks, CHUNK), jnp.int32),
            pltpu.VMEM((CHUNK, c_out), jnp.float32),
            pltpu.VMEM((CHUNK, 8), jnp.float32),
            pltpu.VMEM((rows_per_tile, c_out), jnp.float32),
            pltpu.VMEM((rows_per_tile, 8), jnp.float32),
            pltpu.SemaphoreType.DMA,
        ],
        compiler_params=pltpu.CompilerParams(use_tc_tiling_on_sc=False),
    )
    s_part, c_part = edge_kernel(b_nodes, src_pad, dst_pad, ones_rows, z32, z8)

    out_pad = pl.pallas_call(
        _finalize_body,
        out_shape=jax.ShapeDtypeStruct((n_pad, c_out), jnp.float32),
    )(a_nodes, b.reshape(1, c_out), s_part, c_part)

    return out_pad[:n]


# speedup vs baseline: 26.0028x; 1.2007x over previous
import functools
import jax
import jax.numpy as jnp
from jax import lax
from jax.experimental import pallas as pl
from jax.experimental.pallas import tpu as pltpu
from jax.experimental.pallas import tpu_sc as plsc

NEG_SLOPE = 0.3
NC, NS = 2, 16
NW = NC * NS
CHUNK = 80
DEPTH = 4
AHEAD = 2


def _matmul_body(x_ref, w_ref, a_ref, b_ref, *, c_out):
    y = jnp.dot(x_ref[...], w_ref[...], preferred_element_type=jnp.float32)
    a_ref[...] = y[:, :c_out]
    b_ref[...] = y[:, c_out:]


def _finalize_body(a_ref, bias_ref, s_ref, c_ref, o_ref):
    s = s_ref[0] + s_ref[1]
    cnt = c_ref[0, :, 0:1] + c_ref[1, :, 0:1]
    pre = jnp.where(cnt > 0.0, a_ref[...] + bias_ref[...], 0.0)
    pre = pre + s / jnp.maximum(cnt, 1.0)
    o_ref[...] = jnp.where(pre >= 0.0, pre, NEG_SLOPE * pre)


def _edge_body(b_hbm, src_hbm, dst_hbm, ones_hbm, z32_hbm, z8_hbm,
               s_out, c_out,
               b_sp, s_sp, c_sp,
               src_v, dst_v, buf_v, ones_v, zb32_v, zb8_v,
               gsem, ssem, osem,
               *, n_pad, n_chunks):
    rows_per_tile = n_pad // NS
    c = lax.axis_index("c")
    s = lax.axis_index("s")
    wid = c * NS + s
    base = s * rows_per_tile
    sl = pl.ds(base, rows_per_tile)

    pltpu.sync_copy(z32_hbm, zb32_v)
    pltpu.sync_copy(z8_hbm, zb8_v)
    pltpu.sync_copy(zb32_v, s_sp.at[sl])
    pltpu.sync_copy(zb8_v, c_sp.at[sl])
    pltpu.sync_copy(b_hbm.at[sl], b_sp.at[sl])

    pltpu.sync_copy(src_hbm.at[wid], src_v)
    pltpu.sync_copy(dst_hbm.at[wid], dst_v)
    pltpu.sync_copy(ones_hbm, ones_v)
    plsc.subcore_barrier()

    def fire_gather(j, slot):
        pltpu.async_copy(b_sp.at[src_v.at[j]], buf_v.at[slot], gsem.at[slot])

    def wait_gather(j, slot):
        pltpu.make_async_copy(b_sp.at[src_v.at[j]], buf_v.at[slot],
                              gsem.at[slot]).wait()

    def fire_scatters(j, slot):
        pltpu.async_copy(buf_v.at[slot], s_sp.at[dst_v.at[j]], ssem.at[slot],
                         add=True)
        pltpu.async_copy(ones_v, c_sp.at[dst_v.at[j]], osem.at[slot], add=True)

    def wait_scatter(j, slot):
        pltpu.make_async_copy(buf_v.at[slot], s_sp.at[dst_v.at[j]],
                              ssem.at[slot]).wait()

    def wait_ones(j, slot):
        pltpu.make_async_copy(ones_v, c_sp.at[dst_v.at[j]],
                              osem.at[slot]).wait()

    fire_gather(0, 0)
    fire_gather(1, 1)
    for j in range(AHEAD):
        wait_gather(j, j % DEPTH)
        fire_gather(j + AHEAD, (j + AHEAD) % DEPTH)
        fire_scatters(j, j % DEPTH)

    def body(j, carry):
        sg = lax.rem(j + AHEAD, DEPTH)
        wait_scatter(j - AHEAD, sg)
        fire_gather(j + AHEAD, sg)
        slot = lax.rem(j, DEPTH)
        wait_gather(j, slot)
        fire_scatters(j, slot)
        return carry

    lax.fori_loop(AHEAD, n_chunks - AHEAD, body, 0)

    for j in range(n_chunks - AHEAD, n_chunks):
        wait_scatter(j - AHEAD, (j + AHEAD) % DEPTH)
        wait_gather(j, j % DEPTH)
        fire_scatters(j, j % DEPTH)

    for j in range(n_chunks - AHEAD, n_chunks):
        wait_scatter(j, j % DEPTH)
    for slot in range(DEPTH):
        n_fired = len([k for k in range(n_chunks) if k % DEPTH == slot])
        for _ in range(n_fired):
            wait_ones(0, slot)

    plsc.subcore_barrier()

    pltpu.sync_copy(s_sp.at[sl], s_out.at[c, sl])
    pltpu.sync_copy(c_sp.at[sl], c_out.at[c, sl])


def kernel(features, neighborhood_source, neighborhood_target, W, b):
    n, c_in = features.shape
    e = neighborhood_source.shape[0]
    c_out = W.shape[0]

    n_pad = ((n + NS * 16 - 1) // (NS * 16)) * (NS * 16)
    rows_per_tile = n_pad // NS
    assert e % (NW * CHUNK) == 0
    n_chunks = e // (NW * CHUNK)

    w_loc, w_nb = W[:, :c_in], W[:, c_in:]
    w_cat = jnp.concatenate([w_loc - w_nb, w_nb], axis=0).T
    f_pad = jnp.pad(features, ((0, n_pad - n), (0, 0)))

    src_r = neighborhood_source.reshape(NW, n_chunks, CHUNK)
    dst_r = neighborhood_target.reshape(NW, n_chunks, CHUNK)

    ones_rows = jnp.ones((CHUNK, 8), jnp.float32)
    z32 = jnp.zeros((rows_per_tile, c_out), jnp.float32)
    z8 = jnp.zeros((rows_per_tile, 8), jnp.float32)

    a_nodes, b_nodes = pl.pallas_call(
        functools.partial(_matmul_body, c_out=c_out),
        out_shape=(jax.ShapeDtypeStruct((n_pad, c_out), jnp.float32),
                   jax.ShapeDtypeStruct((n_pad, c_out), jnp.float32)),
    )(f_pad, w_cat)

    mesh = plsc.VectorSubcoreMesh(core_axis_name="c", subcore_axis_name="s",
                                  num_cores=NC, num_subcores=NS)
    edge_kernel = pl.kernel(
        functools.partial(_edge_body, n_pad=n_pad, n_chunks=n_chunks),
        out_type=(jax.ShapeDtypeStruct((NC, n_pad, c_out), jnp.float32),
                  jax.ShapeDtypeStruct((NC, n_pad, 8), jnp.float32)),
        mesh=mesh,
        scratch_types=[
            pltpu.VMEM_SHARED((n_pad, c_out), jnp.float32),
            pltpu.VMEM_SHARED((n_pad, c_out), jnp.float32),
            pltpu.VMEM_SHARED((n_pad, 8), jnp.float32),
            pltpu.VMEM((n_chunks, CHUNK), jnp.int32),
            pltpu.VMEM((n_chunks, CHUNK), jnp.int32),
            pltpu.VMEM((DEPTH, CHUNK, c_out), jnp.float32),
            pltpu.VMEM((CHUNK, 8), jnp.float32),
            pltpu.VMEM((rows_per_tile, c_out), jnp.float32),
            pltpu.VMEM((rows_per_tile, 8), jnp.float32),
            pltpu.SemaphoreType.DMA((DEPTH,)),
            pltpu.SemaphoreType.DMA((DEPTH,)),
            pltpu.SemaphoreType.DMA((DEPTH,)),
        ],
        compiler_params=pltpu.CompilerParams(use_tc_tiling_on_sc=False),
    )
    s_part, c_part = edge_kernel(b_nodes, src_r, dst_r, ones_rows, z32, z8)

    out_pad = pl.pallas_call(
        _finalize_body,
        out_shape=jax.ShapeDtypeStruct((n_pad, c_out), jnp.float32),
    )(a_nodes, b.reshape(1, c_out), s_part, c_part)

    return out_pad[:n]


# speedup vs baseline: 28.2463x; 1.0863x over previous
import functools
import jax
import jax.numpy as jnp
from jax import lax
from jax.experimental import pallas as pl
from jax.experimental.pallas import tpu as pltpu
from jax.experimental.pallas import tpu_sc as plsc

NEG_SLOPE = 0.3
NC, NS = 2, 16
NW = NC * NS
CHUNK = 80
DEPTH = 4
AHEAD = 2
L = 16


def _matmul_body(x_ref, w_ref, ab_ref, *, c_out):
    y = jnp.dot(x_ref[...], w_ref[...], preferred_element_type=jnp.float32)
    ab_ref[0] = y[:, :c_out]
    ab_ref[1] = y[:, c_out:]


def _edge_body(ab_hbm, src_hbm, dst_hbm, ones_hbm, z32_hbm, z1_hbm,
               s_out, c_out,
               b_sp, s_sp, c_sp,
               src_v, dst_v, buf_v, ones_v,
               gsem, ssem, osem,
               *, n_pad, n_chunks):
    rows_per_tile = n_pad // NS
    c = lax.axis_index("c")
    s = lax.axis_index("s")
    wid = c * NS + s
    sl = pl.ds(s * rows_per_tile, rows_per_tile)

    pltpu.sync_copy(z32_hbm, s_sp.at[sl])
    pltpu.sync_copy(z1_hbm, c_sp.at[sl])
    pltpu.sync_copy(ab_hbm.at[1, sl], b_sp.at[sl])

    pltpu.sync_copy(src_hbm.at[wid], src_v)
    pltpu.sync_copy(dst_hbm.at[wid], dst_v)
    pltpu.sync_copy(ones_hbm, ones_v)
    plsc.subcore_barrier()

    def fire_gather(j, slot):
        pltpu.async_copy(b_sp.at[src_v.at[j]], buf_v.at[slot], gsem.at[slot])

    def wait_gather(j, slot):
        pltpu.make_async_copy(b_sp.at[src_v.at[j]], buf_v.at[slot],
                              gsem.at[slot]).wait()

    def fire_scatters(j, slot):
        pltpu.async_copy(buf_v.at[slot], s_sp.at[dst_v.at[j]], ssem.at[slot],
                         add=True)
        pltpu.async_copy(ones_v, c_sp.at[dst_v.at[j]], osem.at[slot], add=True)

    def wait_scatter(j, slot):
        pltpu.make_async_copy(buf_v.at[slot], s_sp.at[dst_v.at[j]],
                              ssem.at[slot]).wait()

    def wait_ones(j, slot):
        pltpu.make_async_copy(ones_v, c_sp.at[dst_v.at[j]],
                              osem.at[slot]).wait()

    fire_gather(0, 0)
    fire_gather(1, 1)
    for j in range(AHEAD):
        wait_gather(j, j % DEPTH)
        fire_gather(j + AHEAD, (j + AHEAD) % DEPTH)
        fire_scatters(j, j % DEPTH)

    def body(j, carry):
        sg = lax.rem(j + AHEAD, DEPTH)
        wait_scatter(j - AHEAD, sg)
        fire_gather(j + AHEAD, sg)
        slot = lax.rem(j, DEPTH)
        wait_gather(j, slot)
        fire_scatters(j, slot)
        return carry

    lax.fori_loop(AHEAD, n_chunks - AHEAD, body, 0)

    for j in range(n_chunks - AHEAD, n_chunks):
        wait_scatter(j - AHEAD, (j + AHEAD) % DEPTH)
        wait_gather(j, j % DEPTH)
        fire_scatters(j, j % DEPTH)

    for j in range(n_chunks - AHEAD, n_chunks):
        wait_scatter(j, j % DEPTH)
    for slot in range(DEPTH):
        n_fired = len([k for k in range(n_chunks) if k % DEPTH == slot])
        for _ in range(n_fired):
            wait_ones(0, slot)

    plsc.subcore_barrier()

    pltpu.sync_copy(s_sp.at[sl], s_out.at[c, sl])
    pltpu.sync_copy(c_sp.at[sl], c_out.at[c, sl])


def _final_body(ab_hbm, bias_hbm, s_hbm, c_hbm, out_hbm,
                a_v, s0_v, s1_v, c0_v, c1_v, bias_v, out_v,
                *, n_pad, c_out):
    rows = n_pad // NW
    c = lax.axis_index("c")
    s = lax.axis_index("s")
    wid = c * NS + s
    sl = pl.ds(wid * rows, rows)

    pltpu.sync_copy(ab_hbm.at[0, sl], a_v)
    pltpu.sync_copy(s_hbm.at[0, sl], s0_v)
    pltpu.sync_copy(s_hbm.at[1, sl], s1_v)
    pltpu.sync_copy(c_hbm.at[0, sl], c0_v)
    pltpu.sync_copy(c_hbm.at[1, sl], c1_v)
    pltpu.sync_copy(bias_hbm, bias_v)

    n_half = c_out // L

    def row_block(rb, carry):
        base = rb * L
        cs = pl.ds(base, L)
        cnt16 = c0_v[cs] + c1_v[cs]
        inv16 = 1.0 / jnp.maximum(cnt16, 1.0)
        m16 = jnp.minimum(cnt16, 1.0)
        for rr in range(L):
            r = base + rr
            lane = jnp.full((L,), rr, jnp.int32)
            inv = jnp.take(inv16, lane)
            msk = jnp.take(m16, lane)
            for h in range(n_half):
                hs = pl.ds(h * L, L)
                a_h = a_v[r, hs] + bias_v[hs]
                s_h = s0_v[r, hs] + s1_v[r, hs]
                pre = (a_h) * msk + s_h * inv
                out_v[r, hs] = (jnp.maximum(pre, 0.0)
                                + NEG_SLOPE * jnp.minimum(pre, 0.0))
        return carry

    lax.fori_loop(0, rows // L, row_block, 0)
    pltpu.sync_copy(out_v, out_hbm.at[sl])


def kernel(features, neighborhood_source, neighborhood_target, W, b):
    n, c_in = features.shape
    e = neighborhood_source.shape[0]
    c_out = W.shape[0]

    n_pad = ((n + NW * 8 - 1) // (NW * 8)) * (NW * 8)
    rows_per_tile = n_pad // NS
    assert e % (NW * CHUNK) == 0
    n_chunks = e // (NW * CHUNK)

    w_loc, w_nb = W[:, :c_in], W[:, c_in:]
    w_cat = jnp.concatenate([w_loc - w_nb, w_nb], axis=0).T

    src_r = neighborhood_source.reshape(NW, n_chunks, CHUNK)
    dst_r = neighborhood_target.reshape(NW, n_chunks, CHUNK)

    ones_v = jnp.ones((CHUNK,), jnp.float32)
    z32 = jnp.zeros((rows_per_tile, c_out), jnp.float32)
    z1 = jnp.zeros((rows_per_tile,), jnp.float32)

    f_pad = jnp.pad(features, ((0, n_pad - n), (0, 0)))
    ab = pl.pallas_call(
        functools.partial(_matmul_body, c_out=c_out),
        out_shape=jax.ShapeDtypeStruct((2, n_pad, c_out), jnp.float32),
    )(f_pad, w_cat)

    mesh = plsc.VectorSubcoreMesh(core_axis_name="c", subcore_axis_name="s",
                                  num_cores=NC, num_subcores=NS)
    edge_kernel = pl.kernel(
        functools.partial(_edge_body, n_pad=n_pad, n_chunks=n_chunks),
        out_type=(jax.ShapeDtypeStruct((NC, n_pad, c_out), jnp.float32),
                  jax.ShapeDtypeStruct((NC, n_pad), jnp.float32)),
        mesh=mesh,
        scratch_types=[
            pltpu.VMEM_SHARED((n_pad, c_out), jnp.float32),
            pltpu.VMEM_SHARED((n_pad, c_out), jnp.float32),
            pltpu.VMEM_SHARED((n_pad,), jnp.float32),
            pltpu.VMEM((n_chunks, CHUNK), jnp.int32),
            pltpu.VMEM((n_chunks, CHUNK), jnp.int32),
            pltpu.VMEM((DEPTH, CHUNK, c_out), jnp.float32),
            pltpu.VMEM((CHUNK,), jnp.float32),
            pltpu.SemaphoreType.DMA((DEPTH,)),
            pltpu.SemaphoreType.DMA((DEPTH,)),
            pltpu.SemaphoreType.DMA((DEPTH,)),
        ],
        compiler_params=pltpu.CompilerParams(use_tc_tiling_on_sc=False),
    )
    s_part, c_part = edge_kernel(ab, src_r, dst_r, ones_v, z32, z1)

    final_kernel = pl.kernel(
        functools.partial(_final_body, n_pad=n_pad, c_out=c_out),
        out_type=jax.ShapeDtypeStruct((n_pad, c_out), jnp.float32),
        mesh=plsc.VectorSubcoreMesh(core_axis_name="c", subcore_axis_name="s",
                                    num_cores=NC, num_subcores=NS),
        scratch_types=[
            pltpu.VMEM((n_pad // NW, c_out), jnp.float32),
            pltpu.VMEM((n_pad // NW, c_out), jnp.float32),
            pltpu.VMEM((n_pad // NW, c_out), jnp.float32),
            pltpu.VMEM((n_pad // NW,), jnp.float32),
            pltpu.VMEM((n_pad // NW,), jnp.float32),
            pltpu.VMEM((c_out,), jnp.float32),
            pltpu.VMEM((n_pad // NW, c_out), jnp.float32),
        ],
        compiler_params=pltpu.CompilerParams(use_tc_tiling_on_sc=False),
    )
    out_pad = final_kernel(ab, b, s_part, c_part)

    return out_pad[:n]


# speedup vs baseline: 36.4597x; 1.2908x over previous
import functools
import jax
import jax.numpy as jnp
from jax import lax
from jax.experimental import pallas as pl
from jax.experimental.pallas import tpu as pltpu
from jax.experimental.pallas import tpu_sc as plsc

NEG_SLOPE = 0.3
NC, NS = 2, 16
NW = NC * NS
CHUNK = 80
DEPTH = 6
AHEAD = 3
L = 16


def _matmul_body(x_ref, w_ref, ab_ref):
    c_in = x_ref.shape[1]
    c_out = w_ref.shape[0]
    w = w_ref[...]
    w_loc, w_nb = w[:, :c_in], w[:, c_in:]
    w_cat = jnp.concatenate([w_loc - w_nb, w_nb], axis=0)
    y = lax.dot_general(x_ref[...], w_cat, (((1,), (1,)), ((), ())),
                        preferred_element_type=jnp.float32)
    ab_ref[0] = y[:, :c_out]
    ab_ref[1] = y[:, c_out:]


def _edge_body(ab_hbm, src_hbm, dst_hbm,
               s_out, c_out,
               s_sp, c_sp,
               src_v, dst_v, buf_v, ones_v, z32_v, z1_v,
               gsem, ssem, osem,
               *, n_pad, n_chunks):
    rows_per_tile = n_pad // NS
    nz = rows_per_tile // CHUNK
    c = lax.axis_index("c")
    s = lax.axis_index("s")
    wid = c * NS + s
    base = s * rows_per_tile
    sl = pl.ds(base, rows_per_tile)
    b_hbm = ab_hbm.at[1]

    pltpu.async_copy(src_hbm.at[wid], src_v, gsem.at[0])
    pltpu.async_copy(dst_hbm.at[wid], dst_v, gsem.at[1])

    def fill(i, carry):
        z32_v[i, pl.ds(0, L)] = jnp.zeros((L,), jnp.float32)
        z32_v[i, pl.ds(L, L)] = jnp.zeros((L,), jnp.float32)
        return carry
    lax.fori_loop(0, CHUNK, fill, 0)
    for k in range(CHUNK // L):
        z1_v[pl.ds(k * L, L)] = jnp.zeros((L,), jnp.float32)
        ones_v[pl.ds(k * L, L)] = jnp.ones((L,), jnp.float32)

    for k in range(nz):
        pltpu.async_copy(z32_v, s_sp.at[pl.ds(base + k * CHUNK, CHUNK)],
                         osem.at[0])
        pltpu.async_copy(z1_v, c_sp.at[pl.ds(base + k * CHUNK, CHUNK)],
                         osem.at[1])
    for k in range(nz):
        pltpu.make_async_copy(z32_v, s_sp.at[pl.ds(base, CHUNK)],
                              osem.at[0]).wait()
        pltpu.make_async_copy(z1_v, c_sp.at[pl.ds(base, CHUNK)],
                              osem.at[1]).wait()
    pltpu.make_async_copy(src_hbm.at[wid], src_v, gsem.at[0]).wait()
    pltpu.make_async_copy(dst_hbm.at[wid], dst_v, gsem.at[1]).wait()
    plsc.subcore_barrier()

    def fire_gather(j, slot):
        pltpu.async_copy(b_hbm.at[src_v.at[j]], buf_v.at[slot], gsem.at[slot])

    def wait_gather(j, slot):
        pltpu.make_async_copy(b_hbm.at[src_v.at[j]], buf_v.at[slot],
                              gsem.at[slot]).wait()

    def fire_scatters(j, slot):
        pltpu.async_copy(buf_v.at[slot], s_sp.at[dst_v.at[j]], ssem.at[slot],
                         add=True)
        pltpu.async_copy(ones_v, c_sp.at[dst_v.at[j]], osem.at[slot], add=True)

    def wait_scatter(j, slot):
        pltpu.make_async_copy(buf_v.at[slot], s_sp.at[dst_v.at[j]],
                              ssem.at[slot]).wait()

    def wait_ones(j, slot):
        pltpu.make_async_copy(ones_v, c_sp.at[dst_v.at[j]],
                              osem.at[slot]).wait()

    for j in range(AHEAD):
        fire_gather(j, j % DEPTH)
    for j in range(AHEAD):
        wait_gather(j, j % DEPTH)
        fire_gather(j + AHEAD, (j + AHEAD) % DEPTH)
        fire_scatters(j, j % DEPTH)

    def body(j, carry):
        sg = lax.rem(j + AHEAD, DEPTH)
        wait_scatter(j - AHEAD, sg)
        fire_gather(j + AHEAD, sg)
        slot = lax.rem(j, DEPTH)
        wait_gather(j, slot)
        fire_scatters(j, slot)
        return carry

    lax.fori_loop(AHEAD, n_chunks - AHEAD, body, 0)

    for j in range(n_chunks - AHEAD, n_chunks):
        wait_scatter(j - AHEAD, (j + AHEAD) % DEPTH)
        wait_gather(j, j % DEPTH)
        fire_scatters(j, j % DEPTH)

    for j in range(n_chunks - AHEAD, n_chunks):
        wait_scatter(j, j % DEPTH)
    for slot in range(DEPTH):
        n_fired = len([k for k in range(n_chunks) if k % DEPTH == slot])
        for _ in range(n_fired):
            wait_ones(0, slot)

    plsc.subcore_barrier()

    pltpu.sync_copy(s_sp.at[sl], s_out.at[c, sl])
    pltpu.sync_copy(c_sp.at[sl], c_out.at[c, sl])


def _final_body(ab_hbm, bias_hbm, s_hbm, c_hbm, out_hbm,
                a_v, s0_v, s1_v, c0_v, c1_v, bias_v, out_v, fsem,
                *, n, c_out):
    rows = a_v.shape[0]
    c = lax.axis_index("c")
    s = lax.axis_index("s")
    wid = c * NS + s
    base = jnp.minimum(wid * rows, n - rows)
    sl = pl.ds(base, rows)

    pltpu.async_copy(ab_hbm.at[0, sl], a_v, fsem.at[0])
    pltpu.async_copy(s_hbm.at[0, sl], s0_v, fsem.at[1])
    pltpu.async_copy(s_hbm.at[1, sl], s1_v, fsem.at[2])
    pltpu.async_copy(c_hbm.at[0, sl], c0_v, fsem.at[3])
    pltpu.async_copy(c_hbm.at[1, sl], c1_v, fsem.at[4])
    pltpu.async_copy(bias_hbm, bias_v, fsem.at[5])
    pltpu.make_async_copy(ab_hbm.at[0, sl], a_v, fsem.at[0]).wait()
    pltpu.make_async_copy(s_hbm.at[0, sl], s0_v, fsem.at[1]).wait()
    pltpu.make_async_copy(s_hbm.at[1, sl], s1_v, fsem.at[2]).wait()
    pltpu.make_async_copy(c_hbm.at[0, sl], c0_v, fsem.at[3]).wait()
    pltpu.make_async_copy(c_hbm.at[1, sl], c1_v, fsem.at[4]).wait()
    pltpu.make_async_copy(bias_hbm, bias_v, fsem.at[5]).wait()

    n_half = c_out // L

    def row_block(rb, carry):
        rbase = rb * L
        cs = pl.ds(rbase, L)
        cnt16 = c0_v[cs] + c1_v[cs]
        inv16 = 1.0 / jnp.maximum(cnt16, 1.0)
        m16 = jnp.minimum(cnt16, 1.0)
        for rr in range(L):
            r = rbase + rr
            lane = jnp.full((L,), rr, jnp.int32)
            inv = jnp.take(inv16, lane)
            msk = jnp.take(m16, lane)
            for h in range(n_half):
                hs = pl.ds(h * L, L)
                a_h = a_v[r, hs] + bias_v[hs]
                s_h = s0_v[r, hs] + s1_v[r, hs]
                pre = a_h * msk + s_h * inv
                out_v[r, hs] = (jnp.maximum(pre, 0.0)
                                + NEG_SLOPE * jnp.minimum(pre, 0.0))
        return carry

    lax.fori_loop(0, rows // L, row_block, 0)
    pltpu.sync_copy(out_v, out_hbm.at[sl])


def kernel(features, neighborhood_source, neighborhood_target, W, b):
    n, c_in = features.shape
    e = neighborhood_source.shape[0]
    c_out = W.shape[0]

    n_pad = ((n + NW * 8 - 1) // (NW * 8)) * (NW * 8)
    rows_f = n_pad // NW
    assert e % (NW * CHUNK) == 0
    n_chunks = e // (NW * CHUNK)

    src_r = neighborhood_source.reshape(NW, n_chunks, CHUNK)
    dst_r = neighborhood_target.reshape(NW, n_chunks, CHUNK)

    ab = pl.pallas_call(
        _matmul_body,
        out_shape=jax.ShapeDtypeStruct((2, n, c_out), jnp.float32),
    )(features, W)

    mesh = plsc.VectorSubcoreMesh(core_axis_name="c", subcore_axis_name="s",
                                  num_cores=NC, num_subcores=NS)
    edge_kernel = pl.kernel(
        functools.partial(_edge_body, n_pad=n_pad, n_chunks=n_chunks),
        out_type=(jax.ShapeDtypeStruct((NC, n_pad, c_out), jnp.float32),
                  jax.ShapeDtypeStruct((NC, n_pad), jnp.float32)),
        mesh=mesh,
        scratch_types=[
            pltpu.VMEM_SHARED((n_pad, c_out), jnp.float32),
            pltpu.VMEM_SHARED((n_pad,), jnp.float32),
            pltpu.VMEM((n_chunks, CHUNK), jnp.int32),
            pltpu.VMEM((n_chunks, CHUNK), jnp.int32),
            pltpu.VMEM((DEPTH, CHUNK, c_out), jnp.float32),
            pltpu.VMEM((CHUNK,), jnp.float32),
            pltpu.VMEM((CHUNK, c_out), jnp.float32),
            pltpu.VMEM((CHUNK,), jnp.float32),
            pltpu.SemaphoreType.DMA((DEPTH,)),
            pltpu.SemaphoreType.DMA((DEPTH,)),
            pltpu.SemaphoreType.DMA((DEPTH,)),
        ],
        compiler_params=pltpu.CompilerParams(use_tc_tiling_on_sc=False),
    )
    s_part, c_part = edge_kernel(ab, src_r, dst_r)

    final_kernel = pl.kernel(
        functools.partial(_final_body, n=n, c_out=c_out),
        out_type=jax.ShapeDtypeStruct((n, c_out), jnp.float32),
        mesh=plsc.VectorSubcoreMesh(core_axis_name="c", subcore_axis_name="s",
                                    num_cores=NC, num_subcores=NS),
        scratch_types=[
            pltpu.VMEM((rows_f, c_out), jnp.float32),
            pltpu.VMEM((rows_f, c_out), jnp.float32),
            pltpu.VMEM((rows_f, c_out), jnp.float32),
            pltpu.VMEM((rows_f,), jnp.float32),
            pltpu.VMEM((rows_f,), jnp.float32),
            pltpu.VMEM((c_out,), jnp.float32),
            pltpu.VMEM((rows_f, c_out), jnp.float32),
            pltpu.SemaphoreType.DMA((6,)),
        ],
        compiler_params=pltpu.CompilerParams(use_tc_tiling_on_sc=False),
    )
    return final_kernel(ab, b, s_part, c_part)
